# baseline (speedup 1.0000x reference)
import functools

import jax
import jax.numpy as jnp
from jax import lax
from jax.experimental import pallas as pl
from jax.experimental.pallas import tpu as pltpu
from jax.experimental.pallas import tpu_sc as plsc

N = 50000
E = 800000
EMB = 64
HF = 32
XF = 7
EF = 5
NL = 3

NC = 2
NS = 16
CH = 128
E_PAD = 819200
ROWS2D = E_PAD // CH
EPT = ROWS2D // NS
EPT_EA = ROWS2D // (NC * NS)
CHH = 640
HROWS = E_PAD // CHH
HPT = HROWS // NS

K_EA = 8
G_EA = EPT_EA // K_EA
ACC_ROWS = 50176
STRIPE = ACC_ROWS // NS
OUT_ROWS = 50048
OSTRIPE = OUT_ROWS // NS

BLK = 2000
GRID = N // BLK

def _dot(a, b):
    return jnp.dot(a, b, preferred_element_type=jnp.float32)


def _rbf16(x):
    u = jax.lax.bitcast_convert_type(x, jnp.uint32)
    lsb = (u >> 16) & jnp.uint32(1)
    r = u + jnp.uint32(0x7FFF) + lsb
    return jax.lax.bitcast_convert_type(r & jnp.uint32(0xFFFF0000), jnp.float32)


def _hsum_body(h2, srcdst, zrows, out2, acc, idxv, rows, gsem):
    c = lax.axis_index("c")
    s = lax.axis_index("s")
    pltpu.sync_copy(zrows, acc.at[pl.ds(s * STRIPE, STRIPE)])
    plsc.subcore_barrier()

    hsrc = h2.at[c]

    def gbody(i, carry):
        pltpu.sync_copy(srcdst.at[s * HPT + i], idxv)
        pltpu.async_copy(hsrc.at[idxv.at[0]], rows, gsem).wait()
        pltpu.sync_copy(rows, acc.at[idxv.at[1]], add=True)
        return carry

    lax.fori_loop(0, HPT, gbody, 0)
    plsc.subcore_barrier()
    pltpu.sync_copy(
        acc.at[pl.ds(s * OSTRIPE, OSTRIPE)],
        out2.at[c].at[pl.ds(s * OSTRIPE, OSTRIPE)],
    )


_hsum_call = pl.kernel(
    _hsum_body,
    out_type=jax.ShapeDtypeStruct((NC, OUT_ROWS, HF), jnp.float32),
    mesh=plsc.VectorSubcoreMesh(core_axis_name="c", subcore_axis_name="s"),
    scratch_types=[
        pltpu.VMEM_SHARED((ACC_ROWS, HF), jnp.float32),
        pltpu.VMEM((2, CHH), jnp.int32),
        pltpu.VMEM((CHH, HF), jnp.float32),
        pltpu.SemaphoreType.DMA,
    ],
    compiler_params=pltpu.CompilerParams(use_tc_tiling_on_sc=False),
)


def _ea_body(ea, dst2, zrows8, outseg, acc, dst_v, rows_v, gsem):
    c = lax.axis_index("c")
    s = lax.axis_index("s")
    t = c * NS + s
    pltpu.sync_copy(zrows8, acc.at[pl.ds(s * STRIPE, STRIPE)])
    plsc.subcore_barrier()

    def gbody(g, carry):
        row0 = t * EPT_EA + g * K_EA
        e0 = row0 * CH
        pltpu.sync_copy(dst2.at[pl.ds(row0, K_EA)], dst_v)
        d = pltpu.async_copy(ea.at[pl.ds(e0, K_EA * CH)], rows_v, gsem)
        d.wait()
        for j in range(K_EA):
            pltpu.sync_copy(
                rows_v.at[pl.ds(j * CH, CH)], acc.at[dst_v.at[j]], add=True
            )
        return carry

    lax.fori_loop(0, G_EA, gbody, 0)
    plsc.subcore_barrier()
    pltpu.sync_copy(
        acc.at[pl.ds(s * OSTRIPE, OSTRIPE)],
        outseg.at[c].at[pl.ds(s * OSTRIPE, OSTRIPE)],
    )


_ea_call = pl.kernel(
    _ea_body,
    out_type=jax.ShapeDtypeStruct((NC, OUT_ROWS, 8), jnp.float32),
    mesh=plsc.VectorSubcoreMesh(core_axis_name="c", subcore_axis_name="s"),
    scratch_types=[
        pltpu.VMEM_SHARED((ACC_ROWS, 8), jnp.float32),
        pltpu.VMEM((K_EA, CH), jnp.int32),
        pltpu.VMEM((K_EA * CH, 8), jnp.float32),
        pltpu.SemaphoreType.DMA,
    ],
    compiler_params=pltpu.CompilerParams(use_tc_tiling_on_sc=False),
)


def _h0_kernel(x_ref, w_ref, b_ref, h2_ref):
    h = _dot(x_ref[...], w_ref[...]) + jnp.sum(b_ref[...], axis=0)
    h2_ref[...] = jnp.stack([h[:, :HF], h[:, HF:]], axis=0)


def _h0(x, x_mlp_w, x_mlp_b):
    return pl.pallas_call(
        _h0_kernel,
        grid=(GRID,),
        in_specs=[
            pl.BlockSpec((BLK, XF), lambda i: (i, 0)),
            pl.BlockSpec((XF, EMB), lambda i: (0, 0)),
            pl.BlockSpec((XF, EMB), lambda i: (0, 0)),
        ],
        out_specs=pl.BlockSpec((NC, BLK, HF), lambda i: (0, i, 0)),
        out_shape=jax.ShapeDtypeStruct((NC, N, HF), jnp.float32),
    )(x, x_mlp_w, x_mlp_b)


def _upd_kernel(hs_ref, seg_ref, ew_ref, w1_ref, b1_ref, w2_ref, b2_ref,
                z_ref, st_ref):
    i = pl.program_id(0)
    hs = hs_ref[...]
    agg = jnp.concatenate([hs[0], hs[1]], axis=1)
    seg = seg_ref[...]
    segs = seg[0] + seg[1]
    ew = ew_ref[...]
    for k in range(6):
        agg = agg + segs[:, k:k + 1] * ew[k:k + 1, :]
    t = jnp.maximum(_dot(agg, w1_ref[...]) + b1_ref[...], 0.0)
    z = _dot(t, w2_ref[...]) + b2_ref[...]
    z_ref[...] = z

    @pl.when(i == 0)
    def _():
        st_ref[...] = jnp.zeros_like(st_ref)

    st = st_ref[...]
    upd = jnp.concatenate(
        [
            jnp.sum(z, axis=0, keepdims=True),
            jnp.sum(z * z, axis=0, keepdims=True),
            jnp.zeros((6, EMB), jnp.float32),
        ],
        axis=0,
    )
    st_ref[...] = st + upd


def _upd(hs, seg, ew, w1l, b1l, w2l, b2l):
    return pl.pallas_call(
        _upd_kernel,
        grid=(GRID,),
        in_specs=[
            pl.BlockSpec((NC, BLK, HF), lambda i: (0, i, 0)),
            pl.BlockSpec((NC, BLK, 8), lambda i: (0, i, 0)),
            pl.BlockSpec((8, EMB), lambda i: (0, 0)),
            pl.BlockSpec((EMB, 2 * EMB), lambda i: (0, 0)),
            pl.BlockSpec((1, 2 * EMB), lambda i: (0, 0)),
            pl.BlockSpec((2 * EMB, EMB), lambda i: (0, 0)),
            pl.BlockSpec((1, EMB), lambda i: (0, 0)),
        ],
        out_specs=[
            pl.BlockSpec((BLK, EMB), lambda i: (i, 0)),
            pl.BlockSpec((8, EMB), lambda i: (0, 0)),
        ],
        out_shape=[
            jax.ShapeDtypeStruct((N, EMB), jnp.float32),
            jax.ShapeDtypeStruct((8, EMB), jnp.float32),
        ],
    )(hs, seg, ew, w1l, b1l, w2l, b2l)


def _bn_elu_kernel(z_ref, st_ref, g_ref, b_ref, h2_ref):
    st = st_ref[...]
    mean = st[0:1, :] / N
    var = st[1:2, :] / N - mean * mean
    y = (z_ref[...] - mean) * lax.rsqrt(var + 1e-5) * g_ref[...] + b_ref[...]
    y = jnp.where(y > 0, y, jnp.exp(jnp.minimum(y, 0.0)) - 1.0)
    h2_ref[...] = jnp.stack([y[:, :HF], y[:, HF:]], axis=0)


def _bn_elu(z, st, gl, bl):
    return pl.pallas_call(
        _bn_elu_kernel,
        grid=(GRID,),
        in_specs=[
            pl.BlockSpec((BLK, EMB), lambda i: (i, 0)),
            pl.BlockSpec((8, EMB), lambda i: (0, 0)),
            pl.BlockSpec((1, EMB), lambda i: (0, 0)),
            pl.BlockSpec((1, EMB), lambda i: (0, 0)),
        ],
        out_specs=pl.BlockSpec((NC, BLK, HF), lambda i: (0, i, 0)),
        out_shape=jax.ShapeDtypeStruct((NC, N, HF), jnp.float32),
    )(z, st, gl, bl)


def _bn_last_kernel(z_ref, st_ref, g_ref, b_ref, o_ref):
    st = st_ref[...]
    mean = st[0:1, :] / N
    var = st[1:2, :] / N - mean * mean
    o_ref[...] = (z_ref[...] - mean) * lax.rsqrt(var + 1e-5) * g_ref[...] + b_ref[...]


def _bn_last(z, st, gl, bl):
    return pl.pallas_call(
        _bn_last_kernel,
        grid=(GRID,),
        in_specs=[
            pl.BlockSpec((BLK, EMB), lambda i: (i, 0)),
            pl.BlockSpec((8, EMB), lambda i: (0, 0)),
            pl.BlockSpec((1, EMB), lambda i: (0, 0)),
            pl.BlockSpec((1, EMB), lambda i: (0, 0)),
        ],
        out_specs=pl.BlockSpec((BLK, EMB), lambda i: (i, 0)),
        out_shape=jax.ShapeDtypeStruct((N, EMB), jnp.float32),
    )(z, st, gl, bl)


def kernel(x, edge_index, edge_attr, x_mlp_w, x_mlp_b, edge_w, edge_b,
           w1, b1, w2, b2, bn_g, bn_b):
    pad = E_PAD - E
    src = edge_index[0]
    dst = edge_index[1]
    src_p = jnp.concatenate([src, jnp.zeros((pad,), jnp.int32)])
    dst_p = jnp.concatenate([dst, jnp.full((pad,), N, jnp.int32)])
    dst2 = dst_p.reshape(ROWS2D, CH)
    srcdst = jnp.stack([src_p.reshape(HROWS, CHH), dst_p.reshape(HROWS, CHH)],
                       axis=1)
    ea = jnp.concatenate(
        [_rbf16(edge_attr), jnp.ones((E, 1), jnp.float32),
         jnp.zeros((E, 2), jnp.float32)],
        axis=1,
    )
    ea = jnp.concatenate([ea, jnp.zeros((pad, 8), jnp.float32)], axis=0)
    zeros32 = jnp.zeros((STRIPE, HF), jnp.float32)
    zeros8 = jnp.zeros((STRIPE, 8), jnp.float32)

    seg = _ea_call(ea, dst2, zeros8)
    h2 = _h0(x, x_mlp_w, x_mlp_b)

    out = None
    for l in range(NL):
        hs = _hsum_call(h2, srcdst, zeros32)
        ew = jnp.concatenate(
            [_rbf16(edge_w[l]), edge_b[l][None], jnp.zeros((2, EMB), jnp.float32)],
            axis=0,
        )
        z, st = _upd(hs, seg, ew, w1[l], b1[l][None], w2[l], b2[l][None])
        if l < NL - 1:
            h2 = _bn_elu(z, st, bn_g[l][None], bn_b[l][None])
        else:
            out = _bn_last(z, st, bn_g[l][None], bn_b[l][None])
    return out


# speedup vs baseline: 1.0911x; 1.0911x over previous
import functools

import jax
import jax.numpy as jnp
from jax import lax
from jax.experimental import pallas as pl
from jax.experimental.pallas import tpu as pltpu
from jax.experimental.pallas import tpu_sc as plsc

N = 50000
E = 800000
EMB = 64
HF = 32
XF = 7
EF = 5
NL = 3

NC = 2
NS = 16
CH = 128
E_PAD = 819200
ROWS2D = E_PAD // CH
EPT = ROWS2D // NS
EPT_EA = ROWS2D // (NC * NS)
CHH = 512
HROWS = E_PAD // CHH
HPT = HROWS // NS

K_EA = 8
G_EA = EPT_EA // K_EA
ACC_ROWS = 50176
STRIPE = ACC_ROWS // NS
OUT_ROWS = 50048
OSTRIPE = OUT_ROWS // NS

BLK = 2000
GRID = N // BLK

def _dot(a, b):
    return jnp.dot(a, b, preferred_element_type=jnp.float32)


def _rbf16(x):
    u = jax.lax.bitcast_convert_type(x, jnp.uint32)
    lsb = (u >> 16) & jnp.uint32(1)
    r = u + jnp.uint32(0x7FFF) + lsb
    return jax.lax.bitcast_convert_type(r & jnp.uint32(0xFFFF0000), jnp.float32)


def _hsum_body(h2, srcdst, zrows, out2, acc, idxv, rows, gsem, isem):
    c = lax.axis_index("c")
    s = lax.axis_index("s")
    pltpu.sync_copy(zrows, acc.at[pl.ds(s * STRIPE, STRIPE)])
    plsc.subcore_barrier()

    hsrc = h2.at[c]

    pltpu.sync_copy(srcdst.at[s * HPT], idxv.at[0])

    def outer(o, carry):
        for b in range(2):
            i = 2 * o + b
            nxt = s * HPT + jnp.minimum(i + 1, HPT - 1)
            pd = pltpu.async_copy(srcdst.at[nxt], idxv.at[1 - b], isem)
            pltpu.async_copy(hsrc.at[idxv.at[b].at[0]], rows, gsem).wait()
            pltpu.sync_copy(rows, acc.at[idxv.at[b].at[1]], add=True)
            pd.wait()
        return carry

    lax.fori_loop(0, HPT // 2, outer, 0)
    plsc.subcore_barrier()
    pltpu.sync_copy(
        acc.at[pl.ds(s * OSTRIPE, OSTRIPE)],
        out2.at[c].at[pl.ds(s * OSTRIPE, OSTRIPE)],
    )


_hsum_call = pl.kernel(
    _hsum_body,
    out_type=jax.ShapeDtypeStruct((NC, OUT_ROWS, HF), jnp.float32),
    mesh=plsc.VectorSubcoreMesh(core_axis_name="c", subcore_axis_name="s"),
    scratch_types=[
        pltpu.VMEM_SHARED((ACC_ROWS, HF), jnp.float32),
        pltpu.VMEM((2, 2, CHH), jnp.int32),
        pltpu.VMEM((CHH, HF), jnp.float32),
        pltpu.SemaphoreType.DMA,
        pltpu.SemaphoreType.DMA,
    ],
    compiler_params=pltpu.CompilerParams(use_tc_tiling_on_sc=False),
)


def _ea_body(ea, dst2, zrows8, outseg, acc, dst_v, rows_v, gsem):
    c = lax.axis_index("c")
    s = lax.axis_index("s")
    t = c * NS + s
    pltpu.sync_copy(zrows8, acc.at[pl.ds(s * STRIPE, STRIPE)])
    plsc.subcore_barrier()

    def gbody(g, carry):
        row0 = t * EPT_EA + g * K_EA
        e0 = row0 * CH
        pltpu.sync_copy(dst2.at[pl.ds(row0, K_EA)], dst_v)
        d = pltpu.async_copy(ea.at[pl.ds(e0, K_EA * CH)], rows_v, gsem)
        d.wait()
        for j in range(K_EA):
            pltpu.sync_copy(
                rows_v.at[pl.ds(j * CH, CH)], acc.at[dst_v.at[j]], add=True
            )
        return carry

    lax.fori_loop(0, G_EA, gbody, 0)
    plsc.subcore_barrier()
    pltpu.sync_copy(
        acc.at[pl.ds(s * OSTRIPE, OSTRIPE)],
        outseg.at[c].at[pl.ds(s * OSTRIPE, OSTRIPE)],
    )


_ea_call = pl.kernel(
    _ea_body,
    out_type=jax.ShapeDtypeStruct((NC, OUT_ROWS, 8), jnp.float32),
    mesh=plsc.VectorSubcoreMesh(core_axis_name="c", subcore_axis_name="s"),
    scratch_types=[
        pltpu.VMEM_SHARED((ACC_ROWS, 8), jnp.float32),
        pltpu.VMEM((K_EA, CH), jnp.int32),
        pltpu.VMEM((K_EA * CH, 8), jnp.float32),
        pltpu.SemaphoreType.DMA,
    ],
    compiler_params=pltpu.CompilerParams(use_tc_tiling_on_sc=False),
)


def _h0_kernel(x_ref, w_ref, b_ref, h2_ref):
    h = _dot(x_ref[...], w_ref[...]) + jnp.sum(b_ref[...], axis=0)
    h2_ref[...] = jnp.stack([h[:, :HF], h[:, HF:]], axis=0)


def _h0(x, x_mlp_w, x_mlp_b):
    return pl.pallas_call(
        _h0_kernel,
        grid=(GRID,),
        in_specs=[
            pl.BlockSpec((BLK, XF), lambda i: (i, 0)),
            pl.BlockSpec((XF, EMB), lambda i: (0, 0)),
            pl.BlockSpec((XF, EMB), lambda i: (0, 0)),
        ],
        out_specs=pl.BlockSpec((NC, BLK, HF), lambda i: (0, i, 0)),
        out_shape=jax.ShapeDtypeStruct((NC, N, HF), jnp.float32),
    )(x, x_mlp_w, x_mlp_b)


def _upd_kernel(hs_ref, seg_ref, ew_ref, w1_ref, b1_ref, w2_ref, b2_ref,
                z_ref, st_ref):
    i = pl.program_id(0)
    hs = hs_ref[...]
    agg = jnp.concatenate([hs[0], hs[1]], axis=1)
    seg = seg_ref[...]
    segs = seg[0] + seg[1]
    ew = ew_ref[...]
    for k in range(6):
        agg = agg + segs[:, k:k + 1] * ew[k:k + 1, :]
    t = jnp.maximum(_dot(agg, w1_ref[...]) + b1_ref[...], 0.0)
    z = _dot(t, w2_ref[...]) + b2_ref[...]
    z_ref[...] = z

    @pl.when(i == 0)
    def _():
        st_ref[...] = jnp.zeros_like(st_ref)

    st = st_ref[...]
    upd = jnp.concatenate(
        [
            jnp.sum(z, axis=0, keepdims=True),
            jnp.sum(z * z, axis=0, keepdims=True),
            jnp.zeros((6, EMB), jnp.float32),
        ],
        axis=0,
    )
    st_ref[...] = st + upd


def _upd(hs, seg, ew, w1l, b1l, w2l, b2l):
    return pl.pallas_call(
        _upd_kernel,
        grid=(GRID,),
        in_specs=[
            pl.BlockSpec((NC, BLK, HF), lambda i: (0, i, 0)),
            pl.BlockSpec((NC, BLK, 8), lambda i: (0, i, 0)),
            pl.BlockSpec((8, EMB), lambda i: (0, 0)),
            pl.BlockSpec((EMB, 2 * EMB), lambda i: (0, 0)),
            pl.BlockSpec((1, 2 * EMB), lambda i: (0, 0)),
            pl.BlockSpec((2 * EMB, EMB), lambda i: (0, 0)),
            pl.BlockSpec((1, EMB), lambda i: (0, 0)),
        ],
        out_specs=[
            pl.BlockSpec((BLK, EMB), lambda i: (i, 0)),
            pl.BlockSpec((8, EMB), lambda i: (0, 0)),
        ],
        out_shape=[
            jax.ShapeDtypeStruct((N, EMB), jnp.float32),
            jax.ShapeDtypeStruct((8, EMB), jnp.float32),
        ],
    )(hs, seg, ew, w1l, b1l, w2l, b2l)


def _bn_elu_kernel(z_ref, st_ref, g_ref, b_ref, h2_ref):
    st = st_ref[...]
    mean = st[0:1, :] / N
    var = st[1:2, :] / N - mean * mean
    y = (z_ref[...] - mean) * lax.rsqrt(var + 1e-5) * g_ref[...] + b_ref[...]
    y = jnp.where(y > 0, y, jnp.exp(jnp.minimum(y, 0.0)) - 1.0)
    h2_ref[...] = jnp.stack([y[:, :HF], y[:, HF:]], axis=0)


def _bn_elu(z, st, gl, bl):
    return pl.pallas_call(
        _bn_elu_kernel,
        grid=(GRID,),
        in_specs=[
            pl.BlockSpec((BLK, EMB), lambda i: (i, 0)),
            pl.BlockSpec((8, EMB), lambda i: (0, 0)),
            pl.BlockSpec((1, EMB), lambda i: (0, 0)),
            pl.BlockSpec((1, EMB), lambda i: (0, 0)),
        ],
        out_specs=pl.BlockSpec((NC, BLK, HF), lambda i: (0, i, 0)),
        out_shape=jax.ShapeDtypeStruct((NC, N, HF), jnp.float32),
    )(z, st, gl, bl)


def _bn_last_kernel(z_ref, st_ref, g_ref, b_ref, o_ref):
    st = st_ref[...]
    mean = st[0:1, :] / N
    var = st[1:2, :] / N - mean * mean
    o_ref[...] = (z_ref[...] - mean) * lax.rsqrt(var + 1e-5) * g_ref[...] + b_ref[...]


def _bn_last(z, st, gl, bl):
    return pl.pallas_call(
        _bn_last_kernel,
        grid=(GRID,),
        in_specs=[
            pl.BlockSpec((BLK, EMB), lambda i: (i, 0)),
            pl.BlockSpec((8, EMB), lambda i: (0, 0)),
            pl.BlockSpec((1, EMB), lambda i: (0, 0)),
            pl.BlockSpec((1, EMB), lambda i: (0, 0)),
        ],
        out_specs=pl.BlockSpec((BLK, EMB), lambda i: (i, 0)),
        out_shape=jax.ShapeDtypeStruct((N, EMB), jnp.float32),
    )(z, st, gl, bl)


def kernel(x, edge_index, edge_attr, x_mlp_w, x_mlp_b, edge_w, edge_b,
           w1, b1, w2, b2, bn_g, bn_b):
    pad = E_PAD - E
    src = edge_index[0]
    dst = edge_index[1]
    src_p = jnp.concatenate([src, jnp.zeros((pad,), jnp.int32)])
    dst_p = jnp.concatenate([dst, jnp.full((pad,), N, jnp.int32)])
    dst2 = dst_p.reshape(ROWS2D, CH)
    srcdst = jnp.stack([src_p.reshape(HROWS, CHH), dst_p.reshape(HROWS, CHH)],
                       axis=1)
    ea = jnp.concatenate(
        [_rbf16(edge_attr), jnp.ones((E, 1), jnp.float32),
         jnp.zeros((E, 2), jnp.float32)],
        axis=1,
    )
    ea = jnp.concatenate([ea, jnp.zeros((pad, 8), jnp.float32)], axis=0)
    zeros32 = jnp.zeros((STRIPE, HF), jnp.float32)
    zeros8 = jnp.zeros((STRIPE, 8), jnp.float32)

    seg = _ea_call(ea, dst2, zeros8)
    h2 = _h0(x, x_mlp_w, x_mlp_b)

    out = None
    for l in range(NL):
        hs = _hsum_call(h2, srcdst, zeros32)
        ew = jnp.concatenate(
            [_rbf16(edge_w[l]), edge_b[l][None], jnp.zeros((2, EMB), jnp.float32)],
            axis=0,
        )
        z, st = _upd(hs, seg, ew, w1[l], b1[l][None], w2[l], b2[l][None])
        if l < NL - 1:
            h2 = _bn_elu(z, st, bn_g[l][None], bn_b[l][None])
        else:
            out = _bn_last(z, st, bn_g[l][None], bn_b[l][None])
    return out


# speedup vs baseline: 1.0918x; 1.0007x over previous
import jax
import jax.numpy as jnp
from jax import lax
from jax.experimental import pallas as pl
from jax.experimental.pallas import tpu as pltpu
from jax.experimental.pallas import tpu_sc as plsc

N = 50000
E = 800000
EMB = 64
HF = 32
XF = 7
EF = 5
NL = 3

NC = 2
NS = 16
CH = 128
E_PAD = 819200
ROWS2D = E_PAD // CH
EPT_EA = ROWS2D // (NC * NS)
CHH = 512
HROWS = E_PAD // CHH
HPT = HROWS // NS

K_EA = 8
G_EA = EPT_EA // K_EA
ACC_ROWS = 50176
STRIPE = ACC_ROWS // NS
OUT_ROWS = 50048
OSTRIPE = OUT_ROWS // NS

BLK = 2000
GRID = N // BLK

def _dot(a, b):
    return jnp.dot(a, b, preferred_element_type=jnp.float32)


def _rbf16(x):
    u = jax.lax.bitcast_convert_type(x, jnp.uint32)
    lsb = (u >> 16) & jnp.uint32(1)
    r = u + jnp.uint32(0x7FFF) + lsb
    return jax.lax.bitcast_convert_type(r & jnp.uint32(0xFFFF0000), jnp.float32)


def _hsum_body(h2, srcdst, zrows, out2, acc, idxv, rows, gsem, isem):
    c = lax.axis_index("c")
    s = lax.axis_index("s")
    pltpu.sync_copy(zrows, acc.at[pl.ds(s * STRIPE, STRIPE)])
    plsc.subcore_barrier()

    hsrc = h2.at[c]

    pltpu.sync_copy(srcdst.at[s * HPT], idxv.at[0])

    def outer(o, carry):
        for b in range(2):
            i = 2 * o + b
            nxt = s * HPT + jnp.minimum(i + 1, HPT - 1)
            pd = pltpu.async_copy(srcdst.at[nxt], idxv.at[1 - b], isem)
            pltpu.async_copy(hsrc.at[idxv.at[b].at[0]], rows, gsem).wait()
            pltpu.sync_copy(rows, acc.at[idxv.at[b].at[1]], add=True)
            pd.wait()
        return carry

    lax.fori_loop(0, HPT // 2, outer, 0)
    plsc.subcore_barrier()
    pltpu.sync_copy(
        acc.at[pl.ds(s * OSTRIPE, OSTRIPE)],
        out2.at[c].at[pl.ds(s * OSTRIPE, OSTRIPE)],
    )


_hsum_call = pl.kernel(
    _hsum_body,
    out_type=jax.ShapeDtypeStruct((NC, OUT_ROWS, HF), jnp.float32),
    mesh=plsc.VectorSubcoreMesh(core_axis_name="c", subcore_axis_name="s"),
    scratch_types=[
        pltpu.VMEM_SHARED((ACC_ROWS, HF), jnp.float32),
        pltpu.VMEM((2, 2, CHH), jnp.int32),
        pltpu.VMEM((CHH, HF), jnp.float32),
        pltpu.SemaphoreType.DMA,
        pltpu.SemaphoreType.DMA,
    ],
    compiler_params=pltpu.CompilerParams(use_tc_tiling_on_sc=False),
)


def _ea_body(ea, dst2, zrows8, outseg, acc, dst_v, rows_v, gsem):
    c = lax.axis_index("c")
    s = lax.axis_index("s")
    t = c * NS + s
    pltpu.sync_copy(zrows8, acc.at[pl.ds(s * STRIPE, STRIPE)])
    plsc.subcore_barrier()

    def gbody(g, carry):
        row0 = t * EPT_EA + g * K_EA
        e0 = row0 * CH
        pltpu.sync_copy(dst2.at[pl.ds(row0, K_EA)], dst_v)
        d = pltpu.async_copy(ea.at[pl.ds(e0, K_EA * CH)], rows_v, gsem)
        d.wait()
        for j in range(K_EA):
            pltpu.sync_copy(
                rows_v.at[pl.ds(j * CH, CH)], acc.at[dst_v.at[j]], add=True
            )
        return carry

    lax.fori_loop(0, G_EA, gbody, 0)
    plsc.subcore_barrier()
    pltpu.sync_copy(
        acc.at[pl.ds(s * OSTRIPE, OSTRIPE)],
        outseg.at[c].at[pl.ds(s * OSTRIPE, OSTRIPE)],
    )


_ea_call = pl.kernel(
    _ea_body,
    out_type=jax.ShapeDtypeStruct((NC, OUT_ROWS, 8), jnp.float32),
    mesh=plsc.VectorSubcoreMesh(core_axis_name="c", subcore_axis_name="s"),
    scratch_types=[
        pltpu.VMEM_SHARED((ACC_ROWS, 8), jnp.float32),
        pltpu.VMEM((K_EA, CH), jnp.int32),
        pltpu.VMEM((K_EA * CH, 8), jnp.float32),
        pltpu.SemaphoreType.DMA,
    ],
    compiler_params=pltpu.CompilerParams(use_tc_tiling_on_sc=False),
)


def _h0_kernel(x_ref, w_ref, b_ref, h2_ref):
    h = _dot(x_ref[...], w_ref[...]) + jnp.sum(b_ref[...], axis=0)
    h2_ref[...] = jnp.stack([h[:, :HF], h[:, HF:]], axis=0)


def _h0(x, x_mlp_w, x_mlp_b):
    return pl.pallas_call(
        _h0_kernel,
        grid=(GRID,),
        in_specs=[
            pl.BlockSpec((BLK, XF), lambda i: (i, 0)),
            pl.BlockSpec((XF, EMB), lambda i: (0, 0)),
            pl.BlockSpec((XF, EMB), lambda i: (0, 0)),
        ],
        out_specs=pl.BlockSpec((NC, BLK, HF), lambda i: (0, i, 0)),
        out_shape=jax.ShapeDtypeStruct((NC, N, HF), jnp.float32),
    )(x, x_mlp_w, x_mlp_b)


def _upd_kernel(hs_ref, seg_ref, ew_ref, w1_ref, b1_ref, w2_ref, b2_ref,
                z_ref, st_ref):
    i = pl.program_id(0)
    hs = hs_ref[...]
    agg = jnp.concatenate([hs[0], hs[1]], axis=1)
    seg = seg_ref[...]
    segs = seg[0] + seg[1]
    ew = ew_ref[...]
    for k in range(6):
        agg = agg + segs[:, k:k + 1] * ew[k:k + 1, :]
    t = jnp.maximum(_dot(agg, w1_ref[...]) + b1_ref[...], 0.0)
    z = _dot(t, w2_ref[...]) + b2_ref[...]
    z_ref[...] = z

    @pl.when(i == 0)
    def _():
        st_ref[...] = jnp.zeros_like(st_ref)

    st = st_ref[...]
    upd = jnp.concatenate(
        [
            jnp.sum(z, axis=0, keepdims=True),
            jnp.sum(z * z, axis=0, keepdims=True),
            jnp.zeros((6, EMB), jnp.float32),
        ],
        axis=0,
    )
    st_ref[...] = st + upd


def _upd(hs, seg, ew, w1l, b1l, w2l, b2l):
    return pl.pallas_call(
        _upd_kernel,
        grid=(GRID,),
        in_specs=[
            pl.BlockSpec((NC, BLK, HF), lambda i: (0, i, 0)),
            pl.BlockSpec((NC, BLK, 8), lambda i: (0, i, 0)),
            pl.BlockSpec((8, EMB), lambda i: (0, 0)),
            pl.BlockSpec((EMB, 2 * EMB), lambda i: (0, 0)),
            pl.BlockSpec((1, 2 * EMB), lambda i: (0, 0)),
            pl.BlockSpec((2 * EMB, EMB), lambda i: (0, 0)),
            pl.BlockSpec((1, EMB), lambda i: (0, 0)),
        ],
        out_specs=[
            pl.BlockSpec((BLK, EMB), lambda i: (i, 0)),
            pl.BlockSpec((8, EMB), lambda i: (0, 0)),
        ],
        out_shape=[
            jax.ShapeDtypeStruct((N, EMB), jnp.float32),
            jax.ShapeDtypeStruct((8, EMB), jnp.float32),
        ],
    )(hs, seg, ew, w1l, b1l, w2l, b2l)


def _bn_elu_kernel(z_ref, st_ref, g_ref, b_ref, h2_ref):
    st = st_ref[...]
    mean = st[0:1, :] / N
    var = st[1:2, :] / N - mean * mean
    y = (z_ref[...] - mean) * lax.rsqrt(var + 1e-5) * g_ref[...] + b_ref[...]
    y = jnp.where(y > 0, y, jnp.exp(jnp.minimum(y, 0.0)) - 1.0)
    h2_ref[...] = jnp.stack([y[:, :HF], y[:, HF:]], axis=0)


def _bn_elu(z, st, gl, bl):
    return pl.pallas_call(
        _bn_elu_kernel,
        grid=(GRID,),
        in_specs=[
            pl.BlockSpec((BLK, EMB), lambda i: (i, 0)),
            pl.BlockSpec((8, EMB), lambda i: (0, 0)),
            pl.BlockSpec((1, EMB), lambda i: (0, 0)),
            pl.BlockSpec((1, EMB), lambda i: (0, 0)),
        ],
        out_specs=pl.BlockSpec((NC, BLK, HF), lambda i: (0, i, 0)),
        out_shape=jax.ShapeDtypeStruct((NC, N, HF), jnp.float32),
    )(z, st, gl, bl)


def _bn_last_kernel(z_ref, st_ref, g_ref, b_ref, o_ref):
    st = st_ref[...]
    mean = st[0:1, :] / N
    var = st[1:2, :] / N - mean * mean
    o_ref[...] = (z_ref[...] - mean) * lax.rsqrt(var + 1e-5) * g_ref[...] + b_ref[...]


def _bn_last(z, st, gl, bl):
    return pl.pallas_call(
        _bn_last_kernel,
        grid=(GRID,),
        in_specs=[
            pl.BlockSpec((BLK, EMB), lambda i: (i, 0)),
            pl.BlockSpec((8, EMB), lambda i: (0, 0)),
            pl.BlockSpec((1, EMB), lambda i: (0, 0)),
            pl.BlockSpec((1, EMB), lambda i: (0, 0)),
        ],
        out_specs=pl.BlockSpec((BLK, EMB), lambda i: (i, 0)),
        out_shape=jax.ShapeDtypeStruct((N, EMB), jnp.float32),
    )(z, st, gl, bl)


def kernel(x, edge_index, edge_attr, x_mlp_w, x_mlp_b, edge_w, edge_b,
           w1, b1, w2, b2, bn_g, bn_b):
    pad = E_PAD - E
    src = edge_index[0]
    dst = edge_index[1]
    src_p = jnp.concatenate([src, jnp.zeros((pad,), jnp.int32)])
    dst_p = jnp.concatenate([dst, jnp.full((pad,), N, jnp.int32)])
    dst2 = dst_p.reshape(ROWS2D, CH)
    srcdst = jnp.stack([src_p.reshape(HROWS, CHH), dst_p.reshape(HROWS, CHH)],
                       axis=1)
    ea = jnp.concatenate(
        [_rbf16(edge_attr), jnp.ones((E, 1), jnp.float32),
         jnp.zeros((E, 2), jnp.float32)],
        axis=1,
    )
    ea = jnp.concatenate([ea, jnp.zeros((pad, 8), jnp.float32)], axis=0)
    zeros32 = jnp.zeros((STRIPE, HF), jnp.float32)
    zeros8 = jnp.zeros((STRIPE, 8), jnp.float32)

    seg = _ea_call(ea, dst2, zeros8)
    h2 = _h0(x, x_mlp_w, x_mlp_b)

    out = None
    for l in range(NL):
        hs = _hsum_call(h2, srcdst, zeros32)
        ew = jnp.concatenate(
            [_rbf16(edge_w[l]), edge_b[l][None], jnp.zeros((2, EMB), jnp.float32)],
            axis=0,
        )
        z, st = _upd(hs, seg, ew, w1[l], b1[l][None], w2[l], b2[l][None])
        if l < NL - 1:
            h2 = _bn_elu(z, st, bn_g[l][None], bn_b[l][None])
        else:
            out = _bn_last(z, st, bn_g[l][None], bn_b[l][None])
    return out
